# baseline (speedup 1.0000x reference)
import functools

import jax
import jax.numpy as jnp
from jax import lax
from jax.experimental import pallas as pl
from jax.experimental.pallas import tpu as pltpu
from jax.experimental.pallas import tpu_sc as plsc

_D = 256
_HW = 1024
_NB = 8
_K = 8192
_KBLK_A = 1024
_KBLK_B = 512
_COMMIT = 0.25


def _score_body(x2_ref, x_ref, y_ref, idx_ref, loss_ref, rt_ref, ri_ref, acc_ref):
    b = pl.program_id(0)
    kb = pl.program_id(1)
    nb = pl.num_programs(0)
    nkb = pl.num_programs(1)
    x3 = x_ref[0]
    yb = y_ref[...]
    d = lax.dot_general(yb, x3, (((1,), (0,)), ((), ())),
                        preferred_element_type=jnp.float32)
    y2 = jnp.sum(yb * yb, axis=1, keepdims=True)
    d2 = (x2_ref[0] - 2.0 * d) + y2
    t = jnp.sqrt(jnp.maximum(d2, 0.0))
    m = jnp.min(t, axis=0, keepdims=True)
    kio = lax.broadcasted_iota(jnp.int32, t.shape, 0) + kb * t.shape[0]
    il = jnp.min(jnp.where(t == m, kio, jnp.int32(2147483647)),
                 axis=0, keepdims=True)

    @pl.when(kb == 0)
    def _():
        rt_ref[...] = m
        ri_ref[...] = il

    @pl.when(kb > 0)
    def _():
        rt = rt_ref[...]
        better = m < rt
        rt_ref[...] = jnp.where(better, m, rt)
        ri_ref[...] = jnp.where(better, il, ri_ref[...])

    @pl.when(kb == nkb - 1)
    def _():
        idx_ref[0] = ri_ref[...]
        tmin = rt_ref[...]
        s = jnp.sum(tmin * tmin)

        @pl.when(b == 0)
        def _():
            acc_ref[0] = s

        @pl.when(b > 0)
        def _():
            acc_ref[0] = acc_ref[0] + s

        @pl.when(b == nb - 1)
        def _():
            scale = (1.0 + _COMMIT) / float(nb * x3.shape[0] * x3.shape[1])
            loss_ref[...] = jnp.full((1, 1), acc_ref[0] * scale, jnp.float32)


def _onehot_body(idx_ref, enc_ref, perp_ref, cnt_ref, acc_ref):
    kb = pl.program_id(0)
    b = pl.program_id(1)
    nkb = pl.num_programs(0)
    nb = pl.num_programs(1)
    kblk = enc_ref.shape[1]
    idxr = idx_ref[0]
    kio = lax.broadcasted_iota(jnp.int32, (kblk, idxr.shape[1]), 0) + kb * kblk
    oh = jnp.where(idxr == kio, 1.0, 0.0).astype(jnp.float32)
    enc_ref[0] = oh
    part = jnp.sum(oh, axis=1, keepdims=True)

    @pl.when(b == 0)
    def _():
        cnt_ref[...] = part

    @pl.when(b > 0)
    def _():
        cnt_ref[...] = cnt_ref[...] + part

    @pl.when(b == nb - 1)
    def _():
        p = cnt_ref[...] * (1.0 / float(nb * idxr.shape[1]))
        e = jnp.sum(p * jnp.log(p + 1e-10))

        @pl.when(kb == 0)
        def _():
            acc_ref[0] = e

        @pl.when(kb > 0)
        def _():
            acc_ref[0] = acc_ref[0] + e

        @pl.when(kb == nkb - 1)
        def _():
            perp_ref[...] = jnp.full((1, 1), jnp.exp(-acc_ref[0]), jnp.float32)


def _scores_call(x2, x_dn, codebook):
    nkb = _K // _KBLK_A
    return pl.pallas_call(
        _score_body,
        grid=(_NB, nkb),
        in_specs=[
            pl.BlockSpec((1, 1, _HW), lambda b, kb: (b, 0, 0)),
            pl.BlockSpec((1, _D, _HW), lambda b, kb: (b, 0, 0)),
            pl.BlockSpec((_KBLK_A, _D), lambda b, kb: (kb, 0)),
        ],
        out_specs=[
            pl.BlockSpec((1, 1, _HW), lambda b, kb: (b, 0, 0)),
            pl.BlockSpec((1, 1), lambda b, kb: (0, 0)),
        ],
        out_shape=[
            jax.ShapeDtypeStruct((_NB, 1, _HW), jnp.int32),
            jax.ShapeDtypeStruct((1, 1), jnp.float32),
        ],
        scratch_shapes=[
            pltpu.VMEM((1, _HW), jnp.float32),
            pltpu.VMEM((1, _HW), jnp.int32),
            pltpu.SMEM((1,), jnp.float32),
        ],
        compiler_params=pltpu.CompilerParams(
            dimension_semantics=("arbitrary", "arbitrary")),
    )(x2, x_dn, codebook)


def _onehot_call(idx3):
    nkb = _K // _KBLK_B
    return pl.pallas_call(
        _onehot_body,
        grid=(nkb, _NB),
        in_specs=[
            pl.BlockSpec((1, 1, _HW), lambda kb, b: (b, 0, 0)),
        ],
        out_specs=[
            pl.BlockSpec((1, _KBLK_B, _HW), lambda kb, b: (b, kb, 0)),
            pl.BlockSpec((1, 1), lambda kb, b: (0, 0)),
        ],
        out_shape=[
            jax.ShapeDtypeStruct((_NB, _K, _HW), jnp.float32),
            jax.ShapeDtypeStruct((1, 1), jnp.float32),
        ],
        scratch_shapes=[
            pltpu.VMEM((_KBLK_B, 1), jnp.float32),
            pltpu.SMEM((1,), jnp.float32),
        ],
        compiler_params=pltpu.CompilerParams(
            dimension_semantics=("arbitrary", "arbitrary")),
    )(idx3)


def _sc_gather(codebook, idx2):
    info = plsc.get_sparse_core_info()
    nw = info.num_cores * info.num_subcores
    nrows = idx2.shape[0]
    rpw = nrows // nw
    mesh = plsc.VectorSubcoreMesh(core_axis_name="c", subcore_axis_name="s")

    @functools.partial(
        pl.kernel, mesh=mesh,
        out_type=jax.ShapeDtypeStruct((nrows, 128, _D), jnp.float32),
        scratch_types=[
            pltpu.VMEM((rpw, 128), jnp.int32),
            pltpu.VMEM((rpw, 128, _D), jnp.float32),
            pltpu.SemaphoreType.DMA,
        ],
    )
    def k(table_hbm, idx_hbm, out_hbm, idx_v, rows_v, sem):
        wid = lax.axis_index("s") * info.num_cores + lax.axis_index("c")
        base = wid * rpw
        pltpu.sync_copy(idx_hbm.at[pl.ds(base, rpw)], idx_v)
        copies = [
            pltpu.async_copy(table_hbm.at[idx_v.at[j]], rows_v.at[j], sem)
            for j in range(rpw)
        ]
        for c in copies:
            c.wait()
        pltpu.sync_copy(rows_v, out_hbm.at[pl.ds(base, rpw)])

    return k(codebook, idx2)


def kernel(inputs, codebook):
    b, d, h, w = inputs.shape
    x_dn = inputs.reshape(b, d, h * w)
    flat = jnp.transpose(inputs, (0, 2, 3, 1)).reshape(-1, d)
    x2 = jnp.sum(flat * flat, axis=1).reshape(b, 1, h * w)
    idx3, loss11 = _scores_call(x2, x_dn, codebook)
    enc3, perp11 = _onehot_call(idx3)
    q_rows = _sc_gather(codebook, idx3.reshape(-1, 128))
    q = q_rows.reshape(b, h * w, d).transpose(0, 2, 1).reshape(b, d, h, w)
    quantized_st = inputs + (q - inputs)
    return (quantized_st, loss11[0, 0], perp11[0, 0],
            enc3.reshape(b, _K, h, w))
